# baseline (speedup 1.0000x reference)
import functools

import jax
import jax.numpy as jnp
from jax import lax
from jax.experimental import pallas as pl
from jax.experimental.pallas import tpu as pltpu
from jax.experimental.pallas import tpu_sc as plsc


def kernel(token_embeddings, pos_table):
    if token_embeddings.ndim == 2:
        token_embeddings = token_embeddings[None, :, :]
    B, S, D = token_embeddings.shape

    info = plsc.get_sparse_core_info()
    NC, NS, L = info.num_cores, info.num_subcores, info.num_lanes
    NW = NC * NS
    S_W = S // NW
    R = 16
    NCHUNK = S_W // R
    VEC = D // L
    NBUF = 5
    LEAD = 3
    NITER = NCHUNK * B

    mesh = plsc.VectorSubcoreMesh(core_axis_name="c", subcore_axis_name="s")

    @functools.partial(
        pl.kernel,
        out_type=jax.ShapeDtypeStruct((B, S, D), jnp.float32),
        mesh=mesh,
        scratch_types=[
            pltpu.VMEM((NBUF, R, D), jnp.float32),
            pltpu.VMEM((2, R, D), jnp.float32),
            pltpu.SemaphoreType.DMA((NBUF,)),
            pltpu.SemaphoreType.DMA((NBUF,)),
            pltpu.SemaphoreType.DMA((2,)),
        ],
    )
    def sc_add(tok_hbm, pos_hbm, out_hbm, tok_v, pos_v, sem_in, sem_out, sem_pos):
        wid = lax.axis_index("s") * NC + lax.axis_index("c")
        base = wid * S_W

        def in_copy(g):
            c = g // B
            b = g % B
            row0 = base + c * R
            return pltpu.make_async_copy(
                tok_hbm.at[b, pl.ds(row0, R)], tok_v.at[g % NBUF],
                sem_in.at[g % NBUF])

        def out_copy(g):
            c = g // B
            b = g % B
            row0 = base + c * R
            return pltpu.make_async_copy(
                tok_v.at[g % NBUF], out_hbm.at[b, pl.ds(row0, R)],
                sem_out.at[g % NBUF])

        def pos_copy(c):
            row0 = base + c * R
            return pltpu.make_async_copy(
                pos_hbm.at[pl.ds(row0, R)], pos_v.at[c % 2],
                sem_pos.at[c % 2])

        def issue_in(g):
            in_copy(g).start()

            @pl.when(g % B == 0)
            def _():
                pos_copy(g // B).start()

        for g in range(LEAD):
            issue_in(g)

        def loop_body(g, _):
            c = g // B
            b = g % B

            @pl.when(g + LEAD < NITER)
            def _():
                @pl.when(g + LEAD >= NBUF)
                def _():
                    out_copy(g + LEAD - NBUF).wait()

                issue_in(g + LEAD)

            in_copy(g).wait()

            @pl.when(b == 0)
            def _():
                pos_copy(c).wait()

            sel = g % NBUF
            cp = c % 2

            @plsc.parallel_loop(0, R * VEC, unroll=16)
            def _(i):
                r = i // VEC
                j = (i % VEC) * L
                plsc.addupdate(
                    tok_v.at[sel, r, pl.ds(j, L)], pos_v[cp, r, pl.ds(j, L)])

            out_copy(g).start()
            return 0

        lax.fori_loop(0, NITER, loop_body, 0)

        for k in range(NBUF):
            out_copy(NITER - NBUF + k).wait()

    return sc_add(token_embeddings, pos_table[:S])


# speedup vs baseline: 1.0158x; 1.0158x over previous
import functools

import jax
import jax.numpy as jnp
from jax import lax
from jax.experimental import pallas as pl
from jax.experimental.pallas import tpu as pltpu
from jax.experimental.pallas import tpu_sc as plsc


def kernel(token_embeddings, pos_table):
    if token_embeddings.ndim == 2:
        token_embeddings = token_embeddings[None, :, :]
    B, S, D = token_embeddings.shape

    info = plsc.get_sparse_core_info()
    NC, NS, L = info.num_cores, info.num_subcores, info.num_lanes
    NW = NC * NS
    S_W = S // NW
    R = 8
    NCHUNK = S_W // R
    VEC = D // L
    NBUF = 10
    LEAD = 5
    NITER = NCHUNK * B

    mesh = plsc.VectorSubcoreMesh(core_axis_name="c", subcore_axis_name="s")

    @functools.partial(
        pl.kernel,
        out_type=jax.ShapeDtypeStruct((B, S, D), jnp.float32),
        mesh=mesh,
        scratch_types=[
            pltpu.VMEM((NBUF, R, D), jnp.float32),
            pltpu.VMEM((2, R, D), jnp.float32),
            pltpu.SemaphoreType.DMA((NBUF,)),
            pltpu.SemaphoreType.DMA((NBUF,)),
            pltpu.SemaphoreType.DMA((2,)),
        ],
    )
    def sc_add(tok_hbm, pos_hbm, out_hbm, tok_v, pos_v, sem_in, sem_out, sem_pos):
        wid = lax.axis_index("s") * NC + lax.axis_index("c")
        base = wid * S_W

        def in_copy(g):
            c = g // B
            b = g % B
            row0 = base + c * R
            return pltpu.make_async_copy(
                tok_hbm.at[b, pl.ds(row0, R)], tok_v.at[g % NBUF],
                sem_in.at[g % NBUF])

        def out_copy(g):
            c = g // B
            b = g % B
            row0 = base + c * R
            return pltpu.make_async_copy(
                tok_v.at[g % NBUF], out_hbm.at[b, pl.ds(row0, R)],
                sem_out.at[g % NBUF])

        def pos_copy(c):
            row0 = base + c * R
            return pltpu.make_async_copy(
                pos_hbm.at[pl.ds(row0, R)], pos_v.at[c % 2],
                sem_pos.at[c % 2])

        def issue_in(g):
            in_copy(g).start()

            @pl.when(g % B == 0)
            def _():
                pos_copy(g // B).start()

        for g in range(LEAD):
            issue_in(g)

        def loop_body(g, _):
            c = g // B
            b = g % B

            @pl.when(g + LEAD < NITER)
            def _():
                @pl.when(g + LEAD >= NBUF)
                def _():
                    out_copy(g + LEAD - NBUF).wait()

                issue_in(g + LEAD)

            in_copy(g).wait()

            @pl.when(b == 0)
            def _():
                pos_copy(c).wait()

            sel = g % NBUF
            cp = c % 2

            @plsc.parallel_loop(0, R * VEC, unroll=16)
            def _(i):
                r = i // VEC
                j = (i % VEC) * L
                plsc.addupdate(
                    tok_v.at[sel, r, pl.ds(j, L)], pos_v[cp, r, pl.ds(j, L)])

            out_copy(g).start()
            return 0

        lax.fori_loop(0, NITER, loop_body, 0)

        for k in range(NBUF):
            out_copy(NITER - NBUF + k).wait()

    return sc_add(token_embeddings, pos_table[:S])
